# baseline (speedup 1.0000x reference)
import functools

import jax
import jax.numpy as jnp
from jax import lax
from jax.experimental import pallas as pl
from jax.experimental.pallas import tpu as pltpu
from jax.experimental.pallas import tpu_sc as plsc

_IDX_CHUNK = 128
_L = 16


def _sc_gather_planar(cam_id, rt, B):
    info = plsc.get_sparse_core_info()
    NW = info.num_cores * info.num_subcores
    bpw = B // NW
    n_chunks = bpw // _IDX_CHUNK

    mesh = plsc.VectorSubcoreMesh(core_axis_name="c", subcore_axis_name="s")

    @functools.partial(
        pl.kernel,
        mesh=mesh,
        out_type=jax.ShapeDtypeStruct((8, B), jnp.float32),
        scratch_types=[
            pltpu.VMEM((bpw,), jnp.int32),
            pltpu.VMEM((bpw, 128), jnp.float32),
            pltpu.VMEM((6, bpw), jnp.float32),
            pltpu.SemaphoreType.DMA,
        ],
        compiler_params=pltpu.CompilerParams(
            use_tc_tiling_on_sc=False, needs_layout_passes=False
        ),
    )
    def k(cam_hbm, rt_hbm, g_hbm, idx_v, rows_v, gp_v, sem):
        wid = lax.axis_index("s") * info.num_cores + lax.axis_index("c")
        base = wid * bpw
        pltpu.sync_copy(cam_hbm.at[pl.ds(base, bpw)], idx_v)
        copies = []
        for j in range(n_chunks):
            sl = pl.ds(j * _IDX_CHUNK, _IDX_CHUNK)
            copies.append(
                pltpu.async_copy(rt_hbm.at[idx_v.at[sl]], rows_v.at[sl], sem)
            )
        for c in copies:
            c.wait()
        lane = lax.iota(jnp.int32, _L)
        for comp in range(6):
            col = jnp.full((_L,), comp, jnp.int32)
            for grp in range(bpw // _L):
                row = lane + grp * _L
                gp_v[comp, pl.ds(grp * _L, _L)] = plsc.load_gather(
                    rows_v, [row, col]
                )
        for comp in range(6):
            pltpu.sync_copy(gp_v.at[comp], g_hbm.at[comp, pl.ds(base, bpw)])

    return k(cam_id, rt)


def _pose_body(g_ref, b_ref, o_ref):
    g = g_ref[...]
    rx = g[0:1, :]
    ry = g[1:2, :]
    rz = g[2:3, :]
    tx = g[3:4, :]
    ty = g[4:5, :]
    tz = g[5:6, :]

    sq = rx * rx + ry * ry + rz * rz
    ang = jnp.sqrt(sq + 1e-12)
    inv = 1.0 / ang
    x = rx * inv
    y = ry * inv
    z = rz * inv
    s = jnp.sin(ang)
    c = jnp.cos(ang)
    C = 1.0 - c

    r00 = c + x * x * C
    r01 = x * y * C - z * s
    r02 = x * z * C + y * s
    r10 = y * x * C + z * s
    r11 = c + y * y * C
    r12 = y * z * C - x * s
    r20 = z * x * C - y * s
    r21 = z * y * C + x * s
    r22 = c + z * z * C

    b00 = b_ref[0, 0]
    b01 = b_ref[0, 1]
    b02 = b_ref[0, 2]
    b03 = b_ref[0, 3]
    b10 = b_ref[1, 0]
    b11 = b_ref[1, 1]
    b12 = b_ref[1, 2]
    b13 = b_ref[1, 3]
    b20 = b_ref[2, 0]
    b21 = b_ref[2, 1]
    b22 = b_ref[2, 2]
    b23 = b_ref[2, 3]

    o00 = r00 * b00 + r01 * b10 + r02 * b20
    o01 = r00 * b01 + r01 * b11 + r02 * b21
    o02 = r00 * b02 + r01 * b12 + r02 * b22
    o03 = r00 * b03 + r01 * b13 + r02 * b23 + tx
    o10 = r10 * b00 + r11 * b10 + r12 * b20
    o11 = r10 * b01 + r11 * b11 + r12 * b21
    o12 = r10 * b02 + r11 * b12 + r12 * b22
    o13 = r10 * b03 + r11 * b13 + r12 * b23 + ty
    o20 = r20 * b00 + r21 * b10 + r22 * b20
    o21 = r20 * b01 + r21 * b11 + r22 * b21
    o22 = r20 * b02 + r21 * b12 + r22 * b22
    o23 = r20 * b03 + r21 * b13 + r22 * b23 + tz

    zero = jnp.zeros_like(tx)
    one = jnp.ones_like(tx)
    E = jnp.concatenate(
        [o00, o01, o02, o03, o10, o11, o12, o13,
         o20, o21, o22, o23, zero, zero, zero, one],
        axis=0,
    )
    o_ref[...] = E.T


def _tc_pose(g, base_top, B):
    BLK = 2048
    out = pl.pallas_call(
        _pose_body,
        grid=(B // BLK,),
        in_specs=[
            pl.BlockSpec((8, BLK), lambda i: (0, i)),
            pl.BlockSpec(memory_space=pltpu.SMEM),
        ],
        out_specs=pl.BlockSpec((BLK, 16), lambda i: (i, 0)),
        out_shape=jax.ShapeDtypeStruct((B, 16), jnp.float32),
    )(g, base_top)
    return out.reshape(B, 4, 4)


def _base_top(base_r, base_s, base_t):
    aa = base_r[0]
    sq = jnp.sum(aa * aa)
    ang = jnp.sqrt(sq + 1e-12)
    x, y, z = aa[0] / ang, aa[1] / ang, aa[2] / ang
    s = jnp.sin(ang)
    c = jnp.cos(ang)
    C = 1.0 - c
    R = jnp.stack(
        [
            jnp.stack([c + x * x * C, x * y * C - z * s, x * z * C + y * s]),
            jnp.stack([y * x * C + z * s, c + y * y * C, y * z * C - x * s]),
            jnp.stack([z * x * C - y * s, z * y * C + x * s, c + z * z * C]),
        ]
    )
    Rs = R * base_s[0, 0]
    return jnp.concatenate([Rs, base_t[0][:, None]], axis=1)


def kernel(cam_id, r, t, base_r, base_s, base_t):
    B = cam_id.shape[0]
    rt = jnp.concatenate(
        [r, t, jnp.zeros((r.shape[0], 122), jnp.float32)], axis=1
    )
    g = _sc_gather_planar(cam_id, rt, B)
    base_top = _base_top(base_r, base_s, base_t)
    return _tc_pose(g, base_top, B)


# speedup vs baseline: 1.4042x; 1.4042x over previous
import functools

import jax
import jax.numpy as jnp
from jax import lax
from jax.experimental import pallas as pl
from jax.experimental.pallas import tpu as pltpu
from jax.experimental.pallas import tpu_sc as plsc

_IDX_CHUNK = 128
_L = 16


def _sc_gather_planar(cam_id, rt, B):
    info = plsc.get_sparse_core_info()
    NW = info.num_cores * info.num_subcores
    bpw = B // NW
    n_chunks = bpw // _IDX_CHUNK

    mesh = plsc.VectorSubcoreMesh(core_axis_name="c", subcore_axis_name="s")

    @functools.partial(
        pl.kernel,
        mesh=mesh,
        out_type=jax.ShapeDtypeStruct((8, B), jnp.float32),
        scratch_types=[
            pltpu.VMEM((bpw,), jnp.int32),
            pltpu.VMEM((bpw, 8), jnp.float32),
            pltpu.VMEM((6, bpw), jnp.float32),
            pltpu.SemaphoreType.DMA,
        ],
        compiler_params=pltpu.CompilerParams(
            use_tc_tiling_on_sc=False, needs_layout_passes=False
        ),
    )
    def k(cam_hbm, rt_hbm, g_hbm, idx_v, rows_v, gp_v, sem):
        wid = lax.axis_index("s") * info.num_cores + lax.axis_index("c")
        base = wid * bpw
        pltpu.sync_copy(cam_hbm.at[pl.ds(base, bpw)], idx_v)
        copies = []
        for j in range(n_chunks):
            sl = pl.ds(j * _IDX_CHUNK, _IDX_CHUNK)
            copies.append(
                pltpu.async_copy(rt_hbm.at[idx_v.at[sl]], rows_v.at[sl], sem)
            )
        for c in copies:
            c.wait()
        lane = lax.iota(jnp.int32, _L)
        for comp in range(6):
            col = jnp.full((_L,), comp, jnp.int32)
            for grp in range(bpw // _L):
                row = lane + grp * _L
                gp_v[comp, pl.ds(grp * _L, _L)] = plsc.load_gather(
                    rows_v, [row, col]
                )
        for comp in range(6):
            pltpu.sync_copy(gp_v.at[comp], g_hbm.at[comp, pl.ds(base, bpw)])

    return k(cam_id, rt)


def _pose_body(g_ref, b_ref, o_ref):
    g = g_ref[...]
    rx = g[0:1, :]
    ry = g[1:2, :]
    rz = g[2:3, :]
    tx = g[3:4, :]
    ty = g[4:5, :]
    tz = g[5:6, :]

    sq = rx * rx + ry * ry + rz * rz
    ang = jnp.sqrt(sq + 1e-12)
    inv = 1.0 / ang
    x = rx * inv
    y = ry * inv
    z = rz * inv
    s = jnp.sin(ang)
    c = jnp.cos(ang)
    C = 1.0 - c

    r00 = c + x * x * C
    r01 = x * y * C - z * s
    r02 = x * z * C + y * s
    r10 = y * x * C + z * s
    r11 = c + y * y * C
    r12 = y * z * C - x * s
    r20 = z * x * C - y * s
    r21 = z * y * C + x * s
    r22 = c + z * z * C

    b00 = b_ref[0, 0]
    b01 = b_ref[0, 1]
    b02 = b_ref[0, 2]
    b03 = b_ref[0, 3]
    b10 = b_ref[1, 0]
    b11 = b_ref[1, 1]
    b12 = b_ref[1, 2]
    b13 = b_ref[1, 3]
    b20 = b_ref[2, 0]
    b21 = b_ref[2, 1]
    b22 = b_ref[2, 2]
    b23 = b_ref[2, 3]

    o00 = r00 * b00 + r01 * b10 + r02 * b20
    o01 = r00 * b01 + r01 * b11 + r02 * b21
    o02 = r00 * b02 + r01 * b12 + r02 * b22
    o03 = r00 * b03 + r01 * b13 + r02 * b23 + tx
    o10 = r10 * b00 + r11 * b10 + r12 * b20
    o11 = r10 * b01 + r11 * b11 + r12 * b21
    o12 = r10 * b02 + r11 * b12 + r12 * b22
    o13 = r10 * b03 + r11 * b13 + r12 * b23 + ty
    o20 = r20 * b00 + r21 * b10 + r22 * b20
    o21 = r20 * b01 + r21 * b11 + r22 * b21
    o22 = r20 * b02 + r21 * b12 + r22 * b22
    o23 = r20 * b03 + r21 * b13 + r22 * b23 + tz

    zero = jnp.zeros_like(tx)
    one = jnp.ones_like(tx)
    E = jnp.concatenate(
        [o00, o01, o02, o03, o10, o11, o12, o13,
         o20, o21, o22, o23, zero, zero, zero, one],
        axis=0,
    )
    o_ref[...] = E.T


def _tc_pose(g, base_top, B):
    BLK = 4096
    out = pl.pallas_call(
        _pose_body,
        grid=(B // BLK,),
        in_specs=[
            pl.BlockSpec((8, BLK), lambda i: (0, i)),
            pl.BlockSpec(memory_space=pltpu.SMEM),
        ],
        out_specs=pl.BlockSpec((BLK, 16), lambda i: (i, 0)),
        out_shape=jax.ShapeDtypeStruct((B, 16), jnp.float32),
    )(g, base_top)
    return out.reshape(B, 4, 4)


def _base_top(base_r, base_s, base_t):
    aa = base_r[0]
    sq = jnp.sum(aa * aa)
    ang = jnp.sqrt(sq + 1e-12)
    x, y, z = aa[0] / ang, aa[1] / ang, aa[2] / ang
    s = jnp.sin(ang)
    c = jnp.cos(ang)
    C = 1.0 - c
    R = jnp.stack(
        [
            jnp.stack([c + x * x * C, x * y * C - z * s, x * z * C + y * s]),
            jnp.stack([y * x * C + z * s, c + y * y * C, y * z * C - x * s]),
            jnp.stack([z * x * C - y * s, z * y * C + x * s, c + z * z * C]),
        ]
    )
    Rs = R * base_s[0, 0]
    return jnp.concatenate([Rs, base_t[0][:, None]], axis=1)


def kernel(cam_id, r, t, base_r, base_s, base_t):
    B = cam_id.shape[0]
    rt = jnp.concatenate(
        [r, t, jnp.zeros((r.shape[0], 2), jnp.float32)], axis=1
    )
    g = _sc_gather_planar(cam_id, rt, B)
    base_top = _base_top(base_r, base_s, base_t)
    return _tc_pose(g, base_top, B)
